# Initial kernel scaffold; baseline (speedup 1.0000x reference)
import jax
import jax.numpy as jnp
from jax.experimental import pallas as pl


def kernel(nodes, edges, receivers, W_node, b_node, W_edge, b_edge, W_msg, b_msg, W_upd, b_upd, W_att, b_att, W_o1, b_o1, W_o2, b_o2, W_o3, b_o3):
    raise NotImplementedError("write your pallas kernel here")


# speedup vs baseline: 10.8637x; 10.8637x over previous
import functools

import jax
import jax.numpy as jnp
from jax import lax
from jax.experimental import pallas as pl
from jax.experimental.pallas import tpu as pltpu
from jax.experimental.pallas import tpu_sc as plsc

_N = 10000
_E = 320000
_DF = 128
_H = 64
_STEPS = 3

_NC = 2
_NS = 16
_NW = _NC * _NS
_EPW = _E // _NW
_VPB = 25
_EPB = _VPB * 16
_NB = _EPW // _EPB


def _rne16(x):
  b = lax.bitcast_convert_type(x, jnp.int32)
  t = (b >> 16) & 1
  r = (b + 32767 + t) & jnp.int32(-65536)
  return lax.bitcast_convert_type(r, jnp.float32)


def _vgather(v, idx):
  return lax.gather(
      v,
      idx[:, None],
      lax.GatherDimensionNumbers(
          offset_dims=(), collapsed_slice_dims=(0,), start_index_map=(0,)
      ),
      slice_sizes=(1,),
      mode=lax.GatherScatterMode.PROMISE_IN_BOUNDS,
  )


def _sc_body(ev_hbm, rv_hbm, wbf_hbm, be_hbm, out_hbm, data_v, recv_v, wbf_v,
             be_v, res_v):
  wid = lax.axis_index("s") * _NC + lax.axis_index("c")
  ebase = wid * _EPW
  pltpu.sync_copy(ev_hbm.at[pl.ds(ebase * 4, _EPW * 4)],
                  data_v.at[pl.ds(0, _EPW * 4)])
  pltpu.sync_copy(rv_hbm.at[pl.ds(ebase, _EPW)], recv_v)
  pltpu.sync_copy(wbf_hbm, wbf_v)
  pltpu.sync_copy(be_hbm, be_v)

  lanes = lax.iota(jnp.int32, 16)
  zf = jnp.zeros((16,), jnp.float32)
  for j in range(5):
    res_v[pl.ds(16 * j, 16)] = zf

  def edge_accum(e_local):
    dv = data_v[pl.ds(e_local * 4, 16)]
    fb = [_rne16(jnp.full((16,), dv[k], jnp.float32)) for k in range(4)]
    for j in range(4):
      acc = be_v[pl.ds(16 * j, 16)]
      for k in range(4):
        acc = acc + fb[k] * wbf_v[pl.ds(64 * k + 16 * j, 16)]
      acc = _rne16(acc)
      res_v[pl.ds(16 * j, 16)] = res_v[pl.ds(16 * j, 16)] + acc

  def block_body(b, cf):
    base_b = b * _EPB

    def inner(v, carry):
      cfi, mi = carry
      r = recv_v[pl.ds(base_b + v * 16, 16)]
      w = jnp.where(r == 0, 1.0, 0.0)
      return (cfi + w, mi + w)

    cf2, macc = lax.fori_loop(0, _VPB, inner, (cf, zf))

    m8 = macc + _vgather(macc, (lanes + 8) & 15)
    m4 = m8 + _vgather(m8, (lanes + 4) & 15)
    m2 = m4 + _vgather(m4, (lanes + 2) & 15)
    m1 = m2 + _vgather(m2, (lanes + 1) & 15)

    @pl.when(m1[0] > 0.0)
    def _():
      def vreg_body(v, _):
        base_v = base_b + v * 16
        r = recv_v[pl.ds(base_v, 16)]
        w = jnp.where(r == 0, 1.0, 0.0)
        w8 = w + _vgather(w, (lanes + 8) & 15)
        w4 = w8 + _vgather(w8, (lanes + 4) & 15)
        w2 = w4 + _vgather(w4, (lanes + 2) & 15)
        w1 = w2 + _vgather(w2, (lanes + 1) & 15)

        @pl.when(w1[0] > 0.0)
        def _():
          def emit(l):
            @pl.when(r[l] == 0)
            def _():
              edge_accum(base_v + l)

          for l in range(16):
            emit(l)

        return 0

      lax.fori_loop(0, _VPB, vreg_body, 0)

    return cf2

  cf = lax.fori_loop(0, _NB, block_body, zf)

  c8 = cf + _vgather(cf, (lanes + 8) & 15)
  c4 = c8 + _vgather(c8, (lanes + 4) & 15)
  c2 = c4 + _vgather(c4, (lanes + 2) & 15)
  c1 = c2 + _vgather(c2, (lanes + 1) & 15)
  res_v[pl.ds(64, 16)] = jnp.where(lanes == 0, c1, 0.0)

  pltpu.sync_copy(res_v, out_hbm.at[wid])


@functools.lru_cache(maxsize=1)
def _sc_reduce():
  return pl.kernel(
      _sc_body,
      out_type=jax.ShapeDtypeStruct((_NW, 80), jnp.float32),
      mesh=plsc.VectorSubcoreMesh(
          core_axis_name="c", subcore_axis_name="s", num_cores=_NC,
          num_subcores=_NS,
      ),
      scratch_types=[
          pltpu.VMEM((_EPW * 4 + 16,), jnp.float32),
          pltpu.VMEM((_EPW,), jnp.int32),
          pltpu.VMEM((256,), jnp.float32),
          pltpu.VMEM((64,), jnp.float32),
          pltpu.VMEM((80,), jnp.float32),
      ],
      name="edge_seg_reduce_sc",
  )


def _bfr(x):
  return x.astype(jnp.bfloat16).astype(jnp.float32)


def _bfr_tc(x):
  b = lax.bitcast_convert_type(x, jnp.int32)
  t = (b >> 16) & 1
  r = (b + 32767 + t) & jnp.int32(-65536)
  return lax.bitcast_convert_type(r, jnp.float32)


def _lrelu(x):
  return jnp.where(x >= 0, x, 0.01 * x)


def _dot(a, b):
  return jax.lax.dot_general(
      a, b, (((1,), (0,)), ((), ())),
      preferred_element_type=jnp.float32,
      precision=jax.lax.Precision.HIGHEST,
  )


def _d16(a, b):
  return jnp.dot(a.astype(jnp.bfloat16), b.astype(jnp.bfloat16),
                 preferred_element_type=jnp.float32)


def _tc_body(part_ref, nodes0_ref, wn_ref, bn_ref, wm_ref, bm_ref, wu_ref,
             bu_ref, wo1_ref, bo1_ref, wo2_ref, bo2_ref, wo3_ref, bo3_ref,
             out_ref):
  red = jnp.sum(part_ref[...], axis=0, keepdims=True)
  s_bf = red[:, 0:_H]
  c0 = red[:, _H:_H + 1]

  s_hi = _bfr_tc(s_bf)
  s_mid = _bfr_tc(s_bf - s_hi)
  s_lo = _bfr_tc(s_bf - s_hi - s_mid)

  emb = _d16(nodes0_ref[...], wn_ref[...]) + bn_ref[...]
  for i in range(_STEPS):
    wm = wm_ref[i]
    agg = ((_d16(s_hi, wm) + _d16(s_mid, wm)) + _d16(s_lo, wm)
           + c0 * bm_ref[i])
    cat = jnp.concatenate([_bfr_tc(emb), _bfr_tc(agg)], axis=1)
    emb = _lrelu(_d16(cat, wu_ref[i]) + bu_ref[i])
  x1 = _lrelu(_d16(emb, wo1_ref[...]) + bo1_ref[...])
  x2 = _lrelu(_d16(x1, wo2_ref[...]) + bo2_ref[...])
  h = jnp.sum(x2 * wo3_ref[...], axis=1, keepdims=True) + bo3_ref[...]
  out_ref[...] = jnp.concatenate(
      [h, jnp.zeros((1, 127), jnp.float32)], axis=1)


@functools.lru_cache(maxsize=1)
def _tc_head():
  return pl.pallas_call(
      _tc_body,
      out_shape=jax.ShapeDtypeStruct((1, 128), jnp.float32),
      name="gcbf_head_tc",
  )


def _output_net(x, W_o1, b_o1, W_o2, b_o2, W_o3, b_o3):
  x = jax.nn.leaky_relu(x @ W_o1 + b_o1)
  x = jax.nn.leaky_relu(x @ W_o2 + b_o2)
  return x @ W_o3 + b_o3


def kernel(nodes, edges, receivers, W_node, b_node, W_edge, b_edge, W_msg,
           b_msg, W_upd, b_upd, W_att, b_att, W_o1, b_o1, W_o2, b_o2, W_o3,
           b_o3):
  del W_att, b_att
  ev = edges.reshape(-1)
  wbf = _bfr(W_edge).reshape(-1)
  partials = _sc_reduce()(ev, receivers, wbf, b_edge)

  nodes0 = lax.slice(nodes, (0, 0), (1, _DF))
  out = _tc_head()(
      partials, nodes0, W_node, b_node.reshape(1, _H), W_msg,
      b_msg.reshape(_STEPS, 1, _H), W_upd, b_upd.reshape(_STEPS, 1, _H),
      W_o1, b_o1.reshape(1, _H), W_o2, b_o2.reshape(1, _H // 2),
      W_o3.reshape(1, _H // 2), b_o3.reshape(1, 1),
  )
  h = out[0, 0]

  def cbf_fn(pos):
    mod_nodes = nodes.at[0, 3:6].set(pos)
    emb = (mod_nodes @ W_node + b_node)[0]
    return _output_net(emb, W_o1, b_o1, W_o2, b_o2, W_o3, b_o3)

  drone_pos = nodes[0, 3:6]
  grad_h = jax.grad(lambda pos: cbf_fn(pos).sum())(drone_pos)
  return h, grad_h
